# baseline (speedup 1.0000x reference)
import functools
import math

import jax
import jax.numpy as jnp
from jax import lax
from jax.experimental import pallas as pl
from jax.experimental.pallas import tpu as pltpu
from jax.experimental.pallas import tpu_sc as plsc

_NUM_CORES = 2
_NUM_SUBCORES = 16
_NUM_WORKERS = _NUM_CORES * _NUM_SUBCORES
_LANES = 16
_C = 64
_NBUF = 4


@functools.partial(jax.jit, static_argnames=("pad",))
def _sc_lpe(xf, mk, idx, pe_table, pad):
    n, d = xf.shape
    v = pe_table.shape[0]
    rows_per_tile = v // _NUM_SUBCORES
    scale = math.sqrt(d)
    per_w = n // _NUM_WORKERS
    n_chunks = per_w // _C
    assert n_chunks % _NBUF == 0
    mk2 = mk.reshape(_NUM_WORKERS, per_w)
    idx2 = idx.reshape(_NUM_WORKERS, per_w)
    mesh = plsc.VectorSubcoreMesh(core_axis_name="c", subcore_axis_name="s")

    @functools.partial(
        pl.kernel,
        mesh=mesh,
        out_type=jax.ShapeDtypeStruct((n, d), jnp.float32),
        scratch_types=[
            pltpu.VMEM((per_w,), jnp.int32),
            pltpu.VMEM((per_w,), jnp.int32),
            *[pltpu.VMEM((_C, d), jnp.float32) for _ in range(2 * _NBUF)],
            *[pltpu.SemaphoreType.DMA for _ in range(2 * _NBUF)],
        ],
    )
    def k(x_hbm, mk_hbm, idx_hbm, tab_hbm, out_hbm, idx_v, mk_v, *bufs):
        rows = bufs[0:_NBUF]
        xs = bufs[_NBUF : 2 * _NBUF]
        sin = bufs[2 * _NBUF : 3 * _NBUF]
        sout = bufs[3 * _NBUF : 4 * _NBUF]
        sid = lax.axis_index("s")
        wid = sid * _NUM_CORES + lax.axis_index("c")
        base_w = wid * per_w

        pltpu.sync_copy(idx_hbm.at[wid], idx_v)
        pltpu.sync_copy(mk_hbm.at[wid], mk_v)

        def _pad(i, carry):
            sl = pl.ds(i * _LANES, _LANES)
            idx_v[sl] = jnp.where(mk_v[sl] != 0, pad, jnp.minimum(idx_v[sl], pad))
            return carry

        lax.fori_loop(0, per_w // _LANES, _pad, 0)

        def issue_in(c, b):
            for jj in range(_C // _LANES):
                vec = idx_v[pl.ds(c * _C + jj * _LANES, _LANES)]
                for i in range(_LANES):
                    r = vec[i]
                    pltpu.async_copy(
                        tab_hbm.at[pl.ds(r, 1)],
                        rows[b].at[pl.ds(jj * _LANES + i, 1)],
                        sin[b],
                    )
            pltpu.async_copy(x_hbm.at[pl.ds(base_w + c * _C, _C)], xs[b], sin[b])

        def wait_in(c, b):
            pltpu.make_async_copy(tab_hbm.at[pl.ds(0, _C)], rows[b], sin[b]).wait()
            pltpu.make_async_copy(
                x_hbm.at[pl.ds(base_w + c * _C, _C)], xs[b], sin[b]
            ).wait()

        def issue_out(c, b):
            pltpu.async_copy(rows[b], out_hbm.at[pl.ds(base_w + c * _C, _C)], sout[b])

        def wait_out(c, b):
            pltpu.make_async_copy(
                rows[b], out_hbm.at[pl.ds(base_w + c * _C, _C)], sout[b]
            ).wait()

        for b in range(_NBUF - 1):
            issue_in(b, b)

        @pl.loop(0, n_chunks, step=_NBUF)
        def _main(g):
            for b in range(_NBUF):
                c = g + b
                wait_in(c, b)

                def _fma(i, carry):
                    for j in range(d // _LANES):
                        sl = pl.ds(j * _LANES, _LANES)
                        rows[b][i, sl] = scale * xs[b][i, sl] + rows[b][i, sl]
                    return carry

                lax.fori_loop(0, _C, _fma, 0)

                issue_out(c, b)
                nxt = c + _NBUF - 1
                bp = (b + _NBUF - 1) % _NBUF

                @pl.when(nxt < n_chunks)
                def _():
                    @pl.when(c >= 1)
                    def _():
                        wait_out(c - 1, bp)

                    issue_in(nxt, bp)

        for b in range(_NBUF):
            wait_out(n_chunks - _NBUF + b, b)

    return k(xf, mk2, idx2, pe_table)


def kernel(x, mask, indices, pe_table):
    b, s, d = x.shape
    n = b * s
    v = pe_table.shape[0]
    v_pad = ((v + 8 * _NUM_SUBCORES - 1) // (8 * _NUM_SUBCORES)) * (8 * _NUM_SUBCORES)
    tab = jnp.pad(pe_table, ((0, v_pad - v), (0, 0)))
    xf = x.reshape(n, d)
    mk = mask.reshape(n).astype(jnp.int32)
    idx = indices.reshape(n).astype(jnp.int32)
    out = _sc_lpe(xf, mk, idx, tab, v - 1)
    return out.reshape(b, s, d)


# speedup vs baseline: 2.5444x; 2.5444x over previous
import functools
import math

import jax
import jax.numpy as jnp
from jax import lax
from jax.experimental import pallas as pl
from jax.experimental.pallas import tpu as pltpu
from jax.experimental.pallas import tpu_sc as plsc

_NUM_CORES = 2
_NUM_SUBCORES = 16
_NUM_WORKERS = _NUM_CORES * _NUM_SUBCORES
_LANES = 16
_PG = 4
_CG = 8
_P = 64
_NBUF = 4


@functools.partial(jax.jit, static_argnames=("pad",))
def _sc_lpe(xf, mk, idx, pe_table, pad):
    n = xf.shape[0]
    dc = xf.shape[2]
    d = _CG * dc
    v = pe_table.shape[1] // dc
    scale = math.sqrt(d)
    per_pg = n // _PG
    n_chunks = per_pg // _P
    assert n_chunks % _NBUF == 0
    mesh = plsc.VectorSubcoreMesh(core_axis_name="c", subcore_axis_name="s")

    @functools.partial(
        pl.kernel,
        mesh=mesh,
        out_type=jax.ShapeDtypeStruct((n, _CG, dc), jnp.float32),
        scratch_types=[
            pltpu.VMEM((v * dc,), jnp.float32),
            *[pltpu.VMEM((_P,), jnp.int32) for _ in range(_NBUF)],
            *[pltpu.VMEM((_P,), jnp.int32) for _ in range(_NBUF)],
            *[pltpu.VMEM((_P, dc), jnp.float32) for _ in range(_NBUF)],
            *[pltpu.SemaphoreType.DMA for _ in range(2 * _NBUF)],
        ],
    )
    def k(x_hbm, mk_hbm, idx_hbm, tab_hbm, out_hbm, tab_v, *bufs):
        idxb = bufs[0:_NBUF]
        mkb = bufs[_NBUF : 2 * _NBUF]
        xb = bufs[2 * _NBUF : 3 * _NBUF]
        sin = bufs[3 * _NBUF : 4 * _NBUF]
        sout = bufs[4 * _NBUF : 5 * _NBUF]
        wid = lax.axis_index("s") * _NUM_CORES + lax.axis_index("c")
        pg = wid % _PG
        cg = wid // _PG
        col0 = cg * dc
        pos0 = pg * per_pg

        pltpu.sync_copy(tab_hbm.at[cg], tab_v)

        def issue_in(c, b):
            base = pos0 + c * _P
            pltpu.async_copy(idx_hbm.at[pl.ds(base, _P)], idxb[b], sin[b])
            pltpu.async_copy(mk_hbm.at[pl.ds(base, _P)], mkb[b], sin[b])
            pltpu.async_copy(
                x_hbm.at[pl.ds(base, _P), cg, :], xb[b], sin[b]
            )

        def wait_in(c, b):
            base = pos0 + c * _P
            pltpu.make_async_copy(idx_hbm.at[pl.ds(base, _P)], idxb[b], sin[b]).wait()
            pltpu.make_async_copy(mk_hbm.at[pl.ds(base, _P)], mkb[b], sin[b]).wait()
            pltpu.make_async_copy(
                x_hbm.at[pl.ds(base, _P), cg, :], xb[b], sin[b]
            ).wait()

        def issue_out(c, b):
            base = pos0 + c * _P
            pltpu.async_copy(
                xb[b], out_hbm.at[pl.ds(base, _P), cg, :], sout[b]
            )

        def wait_out(c, b):
            base = pos0 + c * _P
            pltpu.make_async_copy(
                xb[b], out_hbm.at[pl.ds(base, _P), cg, :], sout[b]
            ).wait()

        for b in range(_NBUF - 1):
            issue_in(b, b)

        @pl.loop(0, n_chunks, step=_NBUF)
        def _main(g):
            for b in range(_NBUF):
                c = g + b
                wait_in(c, b)

                def _pad(i, carry):
                    sl = pl.ds(i * _LANES, _LANES)
                    idxb[b][sl] = jnp.where(
                        mkb[b][sl] != 0, pad, jnp.minimum(idxb[b][sl], pad)
                    )
                    return carry

                lax.fori_loop(0, _P // _LANES, _pad, 0)

                def _gfma(g2, carry):
                    vec = idxb[b][pl.ds(g2 * _LANES, _LANES)]
                    for l in range(_LANES):
                        r = vec[l]
                        p = g2 * _LANES + l
                        xb[b][p, :] = (
                            scale * xb[b][p, :] + tab_v[pl.ds(r * dc, dc)]
                        )
                    return carry

                lax.fori_loop(0, _P // _LANES, _gfma, 0)

                issue_out(c, b)
                nxt = c + _NBUF - 1
                bp = (b + _NBUF - 1) % _NBUF

                @pl.when(nxt < n_chunks)
                def _():
                    @pl.when(c >= 1)
                    def _():
                        wait_out(c - 1, bp)

                    issue_in(nxt, bp)

        for b in range(_NBUF):
            wait_out(n_chunks - _NBUF + b, b)

    return k(xf, mk, idx, pe_table)


def kernel(x, mask, indices, pe_table):
    b, s, d = x.shape
    n = b * s
    v = pe_table.shape[0]
    dc = d // _CG
    xf = x.reshape(n, _CG, dc)
    tab = pe_table.reshape(v, _CG, dc).transpose(1, 0, 2).reshape(_CG, v * dc)
    mk = mask.reshape(n).astype(jnp.int32)
    idx = indices.reshape(n).astype(jnp.int32)
    out = _sc_lpe(xf, mk, idx, tab, v - 1)
    return out.reshape(b, s, d)
